# baseline (speedup 1.0000x reference)
import jax
import jax.numpy as jnp
from jax import lax
from jax.experimental import pallas as pl
from jax.experimental.pallas import tpu as pltpu
from jax.experimental.pallas import tpu_sc as plsc

PI = 3.141592653589793
N_ENT = 100000
N_REL = 1000
D = 64
B = 4096
K = 50
GAMMA = 24.0
EPS = 2.0
ER = (GAMMA + EPS) / D

K_PAD = 56
NC, NS = 2, 16
NW = NC * NS
NH = 2
HB = B // NH
ROWS = HB * K_PAD
RPW = ROWS // NW
CHUNK = 112
NCHUNK = RPW // CHUNK
G = 4
SUP = NCHUNK // G
SROWS = G * CHUNK
BPW = HB // NW
BB = 128
TREL = 1008
LANES = 16


def _trig_body(r_ref, out_ref, pk_ref):
    r = r_ref[...] * (PI / ER)
    c = jnp.cos(r)
    s = jnp.sin(r)
    out_ref[...] = jnp.concatenate([c, s], axis=-1)
    cb = lax.bitcast_convert_type(c, jnp.int32)
    sb = lax.bitcast_convert_type(s, jnp.int32)

    def rnd16(bits):
        return (bits + 0x7FFF + ((bits >> 16) & 1)) >> 16

    pk_ref[...] = (rnd16(sb) << 16) | (rnd16(cb) & 0xFFFF)


def _build_trig_tables(rel_pad):
    return pl.pallas_call(
        _trig_body,
        out_shape=(
            jax.ShapeDtypeStruct((TREL, 2 * D), jnp.float32),
            jax.ShapeDtypeStruct((TREL, D), jnp.int32),
        ),
    )(rel_pad)


def _sc_gather_body(ent_hbm, trig_hbm, trigpk_hbm,
                    idx_ent, idx_rel, idx_anc, idx_rq,
                    out_ent, out_trig, out_anc, out_rq,
                    idx_v, idx_s,
                    gsem_a, gsem_b, ssem_a, ssem_b, sem_s):
    wid = lax.axis_index("s") * NC + lax.axis_index("c")

    def smalls(buf):
        for idx_hbm, table, out in ((idx_anc, ent_hbm, out_anc),
                                    (idx_rq, trig_hbm, out_rq)):
            pltpu.sync_copy(idx_hbm.at[wid], idx_s)
            pltpu.async_copy(table.at[idx_s], buf, sem_s).wait()
            pltpu.sync_copy(buf, out.at[pl.ds(wid * BPW, BPW)])

    pl.run_scoped(smalls, pltpu.VMEM((BPW, 2 * D), jnp.float32))

    def ent_phase(buf_a, buf_b):
        pltpu.sync_copy(idx_ent.at[wid], idx_v)
        wbase = wid * RPW

        def fire(s, buf, gsem):
            for bi in range(G):
                pltpu.async_copy(ent_hbm.at[idx_v.at[s * G + bi]],
                                 buf.at[pl.ds(bi * CHUNK, CHUNK)], gsem)

        def drain_gathers(buf, gsem):
            pltpu.make_async_copy(out_ent.at[pl.ds(0, SROWS)], buf,
                                  gsem).wait()

        def scatter(s, buf, ssem):
            pltpu.async_copy(buf,
                             out_ent.at[pl.ds(wbase + s * SROWS, SROWS)],
                             ssem)

        def wait_scatter(buf, ssem):
            pltpu.make_async_copy(buf, out_ent.at[pl.ds(wbase, SROWS)],
                                  ssem).wait()

        fire(0, buf_a, gsem_a)
        fire(1, buf_b, gsem_b)
        drain_gathers(buf_a, gsem_a)
        scatter(0, buf_a, ssem_a)
        drain_gathers(buf_b, gsem_b)
        scatter(1, buf_b, ssem_b)

        def body(s2, _):
            for half, buf, gsem, ssem in ((0, buf_a, gsem_a, ssem_a),
                                          (1, buf_b, gsem_b, ssem_b)):
                s = s2 * 2 + half
                wait_scatter(buf, ssem)
                fire(s, buf, gsem)
                drain_gathers(buf, gsem)
                scatter(s, buf, ssem)
            return 0

        lax.fori_loop(1, SUP // 2, body, 0)
        wait_scatter(buf_a, ssem_a)
        wait_scatter(buf_b, ssem_b)

    pl.run_scoped(ent_phase,
                  pltpu.VMEM((SROWS, 2 * D), jnp.float32),
                  pltpu.VMEM((SROWS, 2 * D), jnp.float32))

    def trig_phase(table_v, stg_a, stg_b):
        pltpu.sync_copy(trigpk_hbm, table_v)
        pltpu.sync_copy(idx_rel.at[wid], idx_v)
        wbase = wid * RPW
        lanes = lax.iota(jnp.int32, LANES)

        def fill(j, stg):
            for g in range(CHUNK // LANES):
                rvec = idx_v[j, pl.ds(g * LANES, LANES)] * D
                slot64 = (g * LANES + lanes) * D
                for w in range(D):
                    v = plsc.load_gather(table_v, [rvec + w])
                    plsc.store_scatter(stg, [slot64 + w], v)

        def wait_stg(stg, ssem):
            pltpu.make_async_copy(stg, out_trig.at[pl.ds(0, CHUNK * D)],
                                  ssem).wait()

        def scat(j, stg, ssem):
            pltpu.async_copy(
                stg,
                out_trig.at[pl.ds((wbase + j * CHUNK) * D, CHUNK * D)],
                ssem)

        fill(0, stg_a)
        scat(0, stg_a, ssem_a)
        fill(1, stg_b)
        scat(1, stg_b, ssem_b)

        def body(j2, _):
            for half, stg, ssem in ((0, stg_a, ssem_a), (1, stg_b, ssem_b)):
                j = j2 * 2 + half
                wait_stg(stg, ssem)
                fill(j, stg)
                scat(j, stg, ssem)
            return 0

        lax.fori_loop(1, NCHUNK // 2, body, 0)
        wait_stg(stg_a, ssem_a)
        wait_stg(stg_b, ssem_b)

    pl.run_scoped(trig_phase,
                  pltpu.VMEM((TREL * D,), jnp.int32),
                  pltpu.VMEM((CHUNK * D,), jnp.int32),
                  pltpu.VMEM((CHUNK * D,), jnp.int32))


def _sc_gather(ent, trig, trigpk, idx_ent, idx_rel, idx_anc, idx_rq):
    mesh = plsc.VectorSubcoreMesh(
        core_axis_name="c", subcore_axis_name="s",
        num_cores=NC, num_subcores=NS)
    fn = pl.kernel(
        _sc_gather_body,
        compiler_params=pltpu.CompilerParams(needs_layout_passes=False),
        out_type=(
            jax.ShapeDtypeStruct((ROWS, 2 * D), jnp.float32),
            jax.ShapeDtypeStruct((ROWS * D,), jnp.int32),
            jax.ShapeDtypeStruct((HB, 2 * D), jnp.float32),
            jax.ShapeDtypeStruct((HB, 2 * D), jnp.float32),
        ),
        mesh=mesh,
        scratch_types=[
            pltpu.VMEM((NCHUNK, CHUNK), jnp.int32),
            pltpu.VMEM((BPW,), jnp.int32),
            pltpu.SemaphoreType.DMA,
            pltpu.SemaphoreType.DMA,
            pltpu.SemaphoreType.DMA,
            pltpu.SemaphoreType.DMA,
            pltpu.SemaphoreType.DMA,
        ],
    )
    return fn(ent, trig, trigpk, idx_ent, idx_rel, idx_anc, idx_rq)


def _main_body(ent_ref, trig_ref, anc_ref, qtrig_ref,
               w1at_ref, w1bt_ref, w2t_ref, b1_ref, b2_ref, out_ref):
    w = trig_ref[...]
    t_cos2 = lax.bitcast_convert_type(w << 16, jnp.float32)
    t_sin2 = lax.bitcast_convert_type((w >> 16) << 16, jnp.float32)
    trig2 = jnp.concatenate([t_cos2, t_sin2], axis=-1)
    tw = jnp.dot(trig2, w1bt_ref[...], preferred_element_type=jnp.float32)
    aq = jnp.dot(qtrig_ref[...], w1at_ref[...],
                 preferred_element_type=jnp.float32)
    act = jnp.maximum(
        tw.reshape(BB, K_PAD, 2 * D) + aq[:, None, :] + b1_ref[...][None],
        0.0)
    out2 = jnp.dot(act.reshape(BB * K_PAD, 2 * D), w2t_ref[...],
                   preferred_element_type=jnp.float32) + b2_ref[...]
    out2_3 = out2.reshape(BB, K_PAD, 2 * D)

    ent3 = ent_ref[...].reshape(BB, K_PAD, 2 * D)
    t_cos = t_cos2.reshape(BB, K_PAD, D)
    t_sin = t_sin2.reshape(BB, K_PAD, D)
    anc = anc_ref[...]
    a_re = anc[:, None, :D]
    a_im = anc[:, None, D:]
    bias_re = a_re * t_cos - a_im * t_sin - ent3[..., :D]
    bias_im = a_re * t_sin + a_im * t_cos - ent3[..., D:]
    prod_re = out2_3[..., :D] * bias_re
    prod_im = out2_3[..., D:] * bias_im
    kmask = lax.broadcasted_iota(jnp.int32, (1, K_PAD, 1), 1) < K
    prod_re = jnp.where(kmask, prod_re, 0.0)
    prod_im = jnp.where(kmask, prod_im, 0.0)
    fr_re = jnp.sum(prod_re, axis=1) * (1.0 / K)
    fr_im = jnp.sum(prod_im, axis=1) * (1.0 / K)

    q = qtrig_ref[...]
    q_cos = q[:, :D]
    q_sin = q[:, D:]
    av_re = anc[:, :D]
    av_im = anc[:, D:]
    out_re = av_re * q_cos - av_im * q_sin + fr_re
    out_im = av_re * q_sin + av_im * q_cos + fr_im
    out_ref[...] = jnp.concatenate([out_re, out_im], axis=-1)


def _main_call(ent_rows, trig_rows, anc, qtrig, w1at, w1bt, w2t, b1, b2):
    grid = (HB // BB,)
    row_spec = pl.BlockSpec((BB * K_PAD, 2 * D), lambda i: (i, 0))
    pk_spec = pl.BlockSpec((BB * K_PAD, D), lambda i: (i, 0))
    q_spec = pl.BlockSpec((BB, 2 * D), lambda i: (i, 0))
    w_spec = pl.BlockSpec((2 * D, 2 * D), lambda i: (0, 0))
    b_spec = pl.BlockSpec((1, 2 * D), lambda i: (0, 0))
    return pl.pallas_call(
        _main_body,
        grid=grid,
        in_specs=[row_spec, pk_spec, q_spec, q_spec,
                  w_spec, w_spec, w_spec, b_spec, b_spec],
        out_specs=q_spec,
        out_shape=jax.ShapeDtypeStruct((HB, 2 * D), jnp.float32),
    )(ent_rows, trig_rows, anc, qtrig, w1at, w1bt, w2t, b1, b2)


@jax.jit
def kernel(entity_embedding, relation_embedding, W1, b1, W2, b2,
           anchors, rel_0, p1_target, p1_rel):
    rel_pad = jnp.zeros((TREL, D), jnp.float32).at[:N_REL + 1].set(
        relation_embedding)
    trig_table, trig_packed = _build_trig_tables(rel_pad)
    trig_packed = trig_packed.reshape(TREL * D)

    def pad_flat(idx):
        idx = jnp.pad(idx.astype(jnp.int32), ((0, 0), (0, K_PAD - K)))
        return idx.reshape(NW, NCHUNK, CHUNK)

    w1at = W1[:, :2 * D].T
    w1bt = W1[:, 2 * D:].T
    w2t = W2.T
    b1r = b1.reshape(1, 2 * D)
    b2r = b2.reshape(1, 2 * D)

    gathered = []
    for h in range(NH):
        qs = slice(h * HB, (h + 1) * HB)
        ent_rows, trig_flat, anc, qtrig = _sc_gather(
            entity_embedding, trig_table, trig_packed,
            pad_flat(p1_target[qs]), pad_flat(p1_rel[qs]),
            anchors[qs].astype(jnp.int32).reshape(NW, BPW),
            rel_0[qs].astype(jnp.int32).reshape(NW, BPW))
        gathered.append((ent_rows, trig_flat.reshape(ROWS, D), anc, qtrig))

    outs = [
        _main_call(ent_rows, trig_rows, anc, qtrig,
                   w1at, w1bt, w2t, b1r, b2r)
        for ent_rows, trig_rows, anc, qtrig in gathered
    ]
    return jnp.concatenate(outs, axis=0)


# speedup vs baseline: 1.1761x; 1.1761x over previous
import jax
import jax.numpy as jnp
from jax import lax
from jax.experimental import pallas as pl
from jax.experimental.pallas import tpu as pltpu
from jax.experimental.pallas import tpu_sc as plsc

PI = 3.141592653589793
N_ENT = 100000
N_REL = 1000
D = 64
B = 4096
K = 50
GAMMA = 24.0
EPS = 2.0
ER = (GAMMA + EPS) / D

K_PAD = 56
NC, NS = 2, 16
NW = NC * NS
ROWS = B * K_PAD
RPW = ROWS // NW
CHUNK = 112
NCHUNK = RPW // CHUNK
G = 4
SUP = NCHUNK // G
SROWS = G * CHUNK
BPW = B // NW
BB = 128
TREL = 1008
LANES = 16


def _trig_body(r_ref, out_ref, pk_ref):
    r = r_ref[...] * (PI / ER)
    c = jnp.cos(r)
    s = jnp.sin(r)
    out_ref[...] = jnp.concatenate([c, s], axis=-1)
    cb = lax.bitcast_convert_type(c, jnp.int32)
    sb = lax.bitcast_convert_type(s, jnp.int32)

    def rnd16(bits):
        return (bits + 0x7FFF + ((bits >> 16) & 1)) >> 16

    pk_ref[...] = (rnd16(sb) << 16) | (rnd16(cb) & 0xFFFF)


def _build_trig_tables(rel_pad):
    return pl.pallas_call(
        _trig_body,
        out_shape=(
            jax.ShapeDtypeStruct((TREL, 2 * D), jnp.float32),
            jax.ShapeDtypeStruct((TREL, D), jnp.int32),
        ),
    )(rel_pad)


def _sc_gather_body(ent_hbm, trig_hbm, trigpk_hbm,
                    idx_ent, idx_rel, idx_anc, idx_rq,
                    out_ent, out_trig, out_anc, out_rq,
                    idx_v, idx_s,
                    gsem_a, gsem_b, ssem_a, ssem_b, tsem_a, tsem_b, sem_s):
    wid = lax.axis_index("s") * NC + lax.axis_index("c")

    def smalls(buf):
        for idx_hbm, table, out in ((idx_anc, ent_hbm, out_anc),
                                    (idx_rq, trig_hbm, out_rq)):
            pltpu.sync_copy(idx_hbm.at[wid], idx_s)
            pltpu.async_copy(table.at[idx_s], buf, sem_s).wait()
            pltpu.sync_copy(buf, out.at[pl.ds(wid * BPW, BPW)])

    pl.run_scoped(smalls, pltpu.VMEM((BPW, 2 * D), jnp.float32))

    def big_phase(table_v, idx_vr, buf_a, buf_b, stg_a):
        pltpu.sync_copy(trigpk_hbm, table_v)
        pltpu.sync_copy(idx_ent.at[wid], idx_v)
        pltpu.sync_copy(idx_rel.at[wid], idx_vr)
        wbase = wid * RPW
        lanes = lax.iota(jnp.int32, LANES)

        def fire(j, buf, gsem):
            pltpu.async_copy(ent_hbm.at[idx_v.at[j]], buf, gsem)

        def drain_gather(buf, gsem):
            pltpu.make_async_copy(out_ent.at[pl.ds(0, CHUNK)], buf,
                                  gsem).wait()

        def scatter(j, buf, ssem):
            pltpu.async_copy(buf, out_ent.at[pl.ds(wbase + j * CHUNK, CHUNK)],
                             ssem)

        def wait_scatter(buf, ssem):
            pltpu.make_async_copy(buf, out_ent.at[pl.ds(wbase, CHUNK)],
                                  ssem).wait()

        def fill(j, stg):
            for g in range(CHUNK // LANES):
                rvec = idx_vr[j, pl.ds(g * LANES, LANES)] * D
                slot64 = (g * LANES + lanes) * D
                for w in range(D):
                    v = plsc.load_gather(table_v, [rvec + w])
                    plsc.store_scatter(stg, [slot64 + w], v)

        def wait_stg(stg, tsem):
            pltpu.make_async_copy(stg, out_trig.at[pl.ds(0, CHUNK * D)],
                                  tsem).wait()

        def scat_stg(j, stg, tsem):
            pltpu.async_copy(
                stg, out_trig.at[pl.ds((wbase + j * CHUNK) * D, CHUNK * D)],
                tsem)

        def step(j, buf, gsem, ssem, first, first_stg):
            if not first:
                wait_scatter(buf, ssem)
            fire(j, buf, gsem)
            if not first_stg:
                wait_stg(stg_a, tsem_a)
            fill(j, stg_a)
            scat_stg(j, stg_a, tsem_a)
            drain_gather(buf, gsem)
            scatter(j, buf, ssem)

        step(0, buf_a, gsem_a, ssem_a, True, True)
        step(1, buf_b, gsem_b, ssem_b, True, False)

        def body(j2, _):
            step(j2 * 2, buf_a, gsem_a, ssem_a, False, False)
            step(j2 * 2 + 1, buf_b, gsem_b, ssem_b, False, False)
            return 0

        lax.fori_loop(1, NCHUNK // 2, body, 0)
        wait_scatter(buf_a, ssem_a)
        wait_scatter(buf_b, ssem_b)
        wait_stg(stg_a, tsem_a)

    pl.run_scoped(big_phase,
                  pltpu.VMEM((TREL * D,), jnp.int32),
                  pltpu.VMEM((NCHUNK, CHUNK), jnp.int32),
                  pltpu.VMEM((CHUNK, 2 * D), jnp.float32),
                  pltpu.VMEM((CHUNK, 2 * D), jnp.float32),
                  pltpu.VMEM((CHUNK * D,), jnp.int32))


def _sc_gather(ent, trig, trigpk, idx_ent, idx_rel, idx_anc, idx_rq):
    mesh = plsc.VectorSubcoreMesh(
        core_axis_name="c", subcore_axis_name="s",
        num_cores=NC, num_subcores=NS)
    fn = pl.kernel(
        _sc_gather_body,
        compiler_params=pltpu.CompilerParams(needs_layout_passes=False),
        out_type=(
            jax.ShapeDtypeStruct((ROWS, 2 * D), jnp.float32),
            jax.ShapeDtypeStruct((ROWS * D,), jnp.int32),
            jax.ShapeDtypeStruct((B, 2 * D), jnp.float32),
            jax.ShapeDtypeStruct((B, 2 * D), jnp.float32),
        ),
        mesh=mesh,
        scratch_types=[
            pltpu.VMEM((NCHUNK, CHUNK), jnp.int32),
            pltpu.VMEM((BPW,), jnp.int32),
            pltpu.SemaphoreType.DMA,
            pltpu.SemaphoreType.DMA,
            pltpu.SemaphoreType.DMA,
            pltpu.SemaphoreType.DMA,
            pltpu.SemaphoreType.DMA,
            pltpu.SemaphoreType.DMA,
            pltpu.SemaphoreType.DMA,
        ],
    )
    return fn(ent, trig, trigpk, idx_ent, idx_rel, idx_anc, idx_rq)


def _main_body(ent_ref, trig_ref, anc_ref, qtrig_ref,
               w1at_ref, w1bt_ref, w2t_ref, b1_ref, b2_ref, out_ref):
    w = trig_ref[...]
    t_cos2 = lax.bitcast_convert_type(w << 16, jnp.float32)
    t_sin2 = lax.bitcast_convert_type((w >> 16) << 16, jnp.float32)
    trig2 = jnp.concatenate([t_cos2, t_sin2], axis=-1)
    tw = jnp.dot(trig2, w1bt_ref[...], preferred_element_type=jnp.float32)
    aq = jnp.dot(qtrig_ref[...], w1at_ref[...],
                 preferred_element_type=jnp.float32)
    act = jnp.maximum(
        tw.reshape(BB, K_PAD, 2 * D) + aq[:, None, :] + b1_ref[...][None],
        0.0)
    out2 = jnp.dot(act.reshape(BB * K_PAD, 2 * D), w2t_ref[...],
                   preferred_element_type=jnp.float32) + b2_ref[...]
    out2_3 = out2.reshape(BB, K_PAD, 2 * D)

    ent3 = ent_ref[...].reshape(BB, K_PAD, 2 * D)
    t_cos = t_cos2.reshape(BB, K_PAD, D)
    t_sin = t_sin2.reshape(BB, K_PAD, D)
    anc = anc_ref[...]
    a_re = anc[:, None, :D]
    a_im = anc[:, None, D:]
    bias_re = a_re * t_cos - a_im * t_sin - ent3[..., :D]
    bias_im = a_re * t_sin + a_im * t_cos - ent3[..., D:]
    prod_re = out2_3[..., :D] * bias_re
    prod_im = out2_3[..., D:] * bias_im
    kmask = lax.broadcasted_iota(jnp.int32, (1, K_PAD, 1), 1) < K
    prod_re = jnp.where(kmask, prod_re, 0.0)
    prod_im = jnp.where(kmask, prod_im, 0.0)
    fr_re = jnp.sum(prod_re, axis=1) * (1.0 / K)
    fr_im = jnp.sum(prod_im, axis=1) * (1.0 / K)

    q = qtrig_ref[...]
    q_cos = q[:, :D]
    q_sin = q[:, D:]
    av_re = anc[:, :D]
    av_im = anc[:, D:]
    out_re = av_re * q_cos - av_im * q_sin + fr_re
    out_im = av_re * q_sin + av_im * q_cos + fr_im
    out_ref[...] = jnp.concatenate([out_re, out_im], axis=-1)


def _main_call(ent_rows, trig_rows, anc, qtrig, w1at, w1bt, w2t, b1, b2):
    grid = (B // BB,)
    row_spec = pl.BlockSpec((BB * K_PAD, 2 * D), lambda i: (i, 0))
    pk_spec = pl.BlockSpec((BB * K_PAD, D), lambda i: (i, 0))
    q_spec = pl.BlockSpec((BB, 2 * D), lambda i: (i, 0))
    w_spec = pl.BlockSpec((2 * D, 2 * D), lambda i: (0, 0))
    b_spec = pl.BlockSpec((1, 2 * D), lambda i: (0, 0))
    return pl.pallas_call(
        _main_body,
        grid=grid,
        in_specs=[row_spec, pk_spec, q_spec, q_spec,
                  w_spec, w_spec, w_spec, b_spec, b_spec],
        out_specs=q_spec,
        out_shape=jax.ShapeDtypeStruct((B, 2 * D), jnp.float32),
    )(ent_rows, trig_rows, anc, qtrig, w1at, w1bt, w2t, b1, b2)


@jax.jit
def kernel(entity_embedding, relation_embedding, W1, b1, W2, b2,
           anchors, rel_0, p1_target, p1_rel):
    rel_pad = jnp.zeros((TREL, D), jnp.float32).at[:N_REL + 1].set(
        relation_embedding)
    trig_table, trig_packed = _build_trig_tables(rel_pad)
    trig_packed = trig_packed.reshape(TREL * D)

    def pad_flat(idx):
        idx = jnp.pad(idx.astype(jnp.int32), ((0, 0), (0, K_PAD - K)))
        return idx.reshape(NW, NCHUNK, CHUNK)

    idx_ent = pad_flat(p1_target)
    idx_rel = pad_flat(p1_rel)
    idx_anc = anchors.astype(jnp.int32).reshape(NW, BPW)
    idx_rq = rel_0.astype(jnp.int32).reshape(NW, BPW)

    ent_rows, trig_flat, anc, qtrig = _sc_gather(
        entity_embedding, trig_table, trig_packed,
        idx_ent, idx_rel, idx_anc, idx_rq)
    trig_rows = trig_flat.reshape(ROWS, D)

    w1at = W1[:, :2 * D].T
    w1bt = W1[:, 2 * D:].T
    w2t = W2.T
    return _main_call(ent_rows, trig_rows, anc, qtrig,
                      w1at, w1bt, w2t,
                      b1.reshape(1, 2 * D), b2.reshape(1, 2 * D))


# speedup vs baseline: 1.2468x; 1.0601x over previous
import jax
import jax.numpy as jnp
from jax import lax
from jax.experimental import pallas as pl
from jax.experimental.pallas import tpu as pltpu
from jax.experimental.pallas import tpu_sc as plsc

PI = 3.141592653589793
N_ENT = 100000
N_REL = 1000
D = 64
B = 4096
K = 50
GAMMA = 24.0
EPS = 2.0
ER = (GAMMA + EPS) / D

K_PAD = 56
NC, NS = 2, 16
NW = NC * NS
ROWS = B * K_PAD
RPW = ROWS // NW
CHUNK = 112
NCHUNK = RPW // CHUNK
G = 4
SUP = NCHUNK // G
SROWS = G * CHUNK
BPW = B // NW
BB = 128
TREL = 1008
LANES = 16


def _trig_body(r_ref, out_ref, pk_ref):
    r = r_ref[...] * (PI / ER)
    c = jnp.cos(r)
    s = jnp.sin(r)
    out_ref[...] = jnp.concatenate([c, s], axis=-1)
    cb = lax.bitcast_convert_type(c, jnp.int32)
    sb = lax.bitcast_convert_type(s, jnp.int32)

    def rnd16(bits):
        return (bits + 0x7FFF + ((bits >> 16) & 1)) >> 16

    pk_ref[...] = (rnd16(sb) << 16) | (rnd16(cb) & 0xFFFF)


def _build_trig_tables(rel_pad):
    return pl.pallas_call(
        _trig_body,
        out_shape=(
            jax.ShapeDtypeStruct((TREL, 2 * D), jnp.float32),
            jax.ShapeDtypeStruct((TREL, D), jnp.int32),
        ),
    )(rel_pad)


def _sc_gather_body(ent_hbm, trig_hbm, trigpk_hbm,
                    idx_ent, idx_rel, idx_anc, idx_rq,
                    out_ent, out_trig, out_anc, out_rq,
                    idx_v, idx_s,
                    gsem_a, gsem_b, ssem_a, ssem_b, tsem_a, tsem_b, sem_s):
    wid = lax.axis_index("s") * NC + lax.axis_index("c")

    def smalls(buf):
        for idx_hbm, table, out in ((idx_anc, ent_hbm, out_anc),
                                    (idx_rq, trig_hbm, out_rq)):
            pltpu.sync_copy(idx_hbm.at[wid], idx_s)
            pltpu.async_copy(table.at[idx_s], buf, sem_s).wait()
            pltpu.sync_copy(buf, out.at[pl.ds(wid * BPW, BPW)])

    pl.run_scoped(smalls, pltpu.VMEM((BPW, 2 * D), jnp.float32))

    def big_phase(table_v, idx_vr, buf_a, buf_b, stg_a):
        pltpu.sync_copy(trigpk_hbm, table_v)
        pltpu.sync_copy(idx_ent.at[wid], idx_v)
        pltpu.sync_copy(idx_rel.at[wid], idx_vr)
        wbase = wid * RPW
        lanes = lax.iota(jnp.int32, LANES)

        def fire(j, buf, gsem):
            pltpu.async_copy(ent_hbm.at[idx_v.at[j]], buf, gsem)

        def drain_gather(buf, gsem):
            pltpu.make_async_copy(out_ent.at[pl.ds(0, CHUNK)], buf,
                                  gsem).wait()

        def scatter(j, buf, ssem):
            pltpu.async_copy(buf, out_ent.at[pl.ds(wbase + j * CHUNK, CHUNK)],
                             ssem)

        def wait_scatter(buf, ssem):
            pltpu.make_async_copy(buf, out_ent.at[pl.ds(wbase, CHUNK)],
                                  ssem).wait()

        def fill(j, stg):
            for g in range(CHUNK // LANES):
                rvec = idx_vr[j, pl.ds(g * LANES, LANES)] * D
                slot64 = (g * LANES + lanes) * D
                for w in range(D):
                    v = plsc.load_gather(table_v, [rvec + w])
                    plsc.store_scatter(stg, [slot64 + w], v)

        def wait_stg(stg, tsem):
            pltpu.make_async_copy(stg, out_trig.at[pl.ds(0, CHUNK * D)],
                                  tsem).wait()

        def scat_stg(j, stg, tsem):
            pltpu.async_copy(
                stg, out_trig.at[pl.ds((wbase + j * CHUNK) * D, CHUNK * D)],
                tsem)

        def step(j, buf, gsem, ssem, first, first_stg):
            if not first:
                wait_scatter(buf, ssem)
            fire(j, buf, gsem)
            if not first_stg:
                wait_stg(stg_a, tsem_a)
            fill(j, stg_a)
            scat_stg(j, stg_a, tsem_a)
            drain_gather(buf, gsem)
            scatter(j, buf, ssem)

        step(0, buf_a, gsem_a, ssem_a, True, True)
        step(1, buf_b, gsem_b, ssem_b, True, False)

        def body(j2, _):
            step(j2 * 2, buf_a, gsem_a, ssem_a, False, False)
            step(j2 * 2 + 1, buf_b, gsem_b, ssem_b, False, False)
            return 0

        lax.fori_loop(1, NCHUNK // 2, body, 0)
        wait_scatter(buf_a, ssem_a)
        wait_scatter(buf_b, ssem_b)
        wait_stg(stg_a, tsem_a)

    pl.run_scoped(big_phase,
                  pltpu.VMEM((TREL * D,), jnp.int32),
                  pltpu.VMEM((NCHUNK, CHUNK), jnp.int32),
                  pltpu.VMEM((CHUNK, 2 * D), jnp.float32),
                  pltpu.VMEM((CHUNK, 2 * D), jnp.float32),
                  pltpu.VMEM((CHUNK * D,), jnp.int32))


def _sc_gather(ent, trig, trigpk, idx_ent, idx_rel, idx_anc, idx_rq):
    mesh = plsc.VectorSubcoreMesh(
        core_axis_name="c", subcore_axis_name="s",
        num_cores=NC, num_subcores=NS)
    fn = pl.kernel(
        _sc_gather_body,
        compiler_params=pltpu.CompilerParams(needs_layout_passes=False),
        out_type=(
            jax.ShapeDtypeStruct((ROWS, 2 * D), jnp.float32),
            jax.ShapeDtypeStruct((ROWS * D,), jnp.int32),
            jax.ShapeDtypeStruct((B, 2 * D), jnp.float32),
            jax.ShapeDtypeStruct((B, 2 * D), jnp.float32),
        ),
        mesh=mesh,
        scratch_types=[
            pltpu.VMEM((NCHUNK, CHUNK), jnp.int32),
            pltpu.VMEM((BPW,), jnp.int32),
            pltpu.SemaphoreType.DMA,
            pltpu.SemaphoreType.DMA,
            pltpu.SemaphoreType.DMA,
            pltpu.SemaphoreType.DMA,
            pltpu.SemaphoreType.DMA,
            pltpu.SemaphoreType.DMA,
            pltpu.SemaphoreType.DMA,
        ],
    )
    return fn(ent, trig, trigpk, idx_ent, idx_rel, idx_anc, idx_rq)


def _main_body(ent_ref, trig_ref, anc_ref, qtrig_ref,
               w1at_ref, w1bt_ref, w2t_ref, b1_ref, b2_ref, out_ref):
    w = trig_ref[...]
    t_cos2 = lax.bitcast_convert_type(w << 16, jnp.float32)
    t_sin2 = lax.bitcast_convert_type((w >> 16) << 16, jnp.float32)
    trig2 = jnp.concatenate([t_cos2, t_sin2], axis=-1)
    tw = jnp.dot(trig2, w1bt_ref[...], preferred_element_type=jnp.float32)
    aq = jnp.dot(qtrig_ref[...], w1at_ref[...],
                 preferred_element_type=jnp.float32)
    act = jnp.maximum(
        tw.reshape(BB, K_PAD, 2 * D) + aq[:, None, :] + b1_ref[...][None],
        0.0)
    out2 = jnp.dot(act.reshape(BB * K_PAD, 2 * D), w2t_ref[...],
                   preferred_element_type=jnp.float32) + b2_ref[...]
    out2_3 = out2.reshape(BB, K_PAD, 2 * D)

    ccat = jnp.concatenate([t_cos2, t_cos2], axis=-1).reshape(BB, K_PAD, 2 * D)
    scat = jnp.concatenate([-t_sin2, t_sin2], axis=-1).reshape(BB, K_PAD, 2 * D)
    anc = anc_ref[...]
    swp = jnp.concatenate([anc[:, D:], anc[:, :D]], axis=-1)
    bias = (anc[:, None, :] * ccat + swp[:, None, :] * scat
            - ent_ref[...].reshape(BB, K_PAD, 2 * D))
    prod = out2_3 * bias
    kmask = lax.broadcasted_iota(jnp.int32, (1, K_PAD, 1), 1) < K
    fr = jnp.sum(jnp.where(kmask, prod, 0.0), axis=1) * (1.0 / K)

    q = qtrig_ref[...]
    qccat = jnp.concatenate([q[:, :D], q[:, :D]], axis=-1)
    qscat = jnp.concatenate([-q[:, D:], q[:, D:]], axis=-1)
    out_ref[...] = anc * qccat + swp * qscat + fr


def _main_call(ent_rows, trig_rows, anc, qtrig, w1at, w1bt, w2t, b1, b2):
    grid = (B // BB,)
    row_spec = pl.BlockSpec((BB * K_PAD, 2 * D), lambda i: (i, 0))
    pk_spec = pl.BlockSpec((BB * K_PAD, D), lambda i: (i, 0))
    q_spec = pl.BlockSpec((BB, 2 * D), lambda i: (i, 0))
    w_spec = pl.BlockSpec((2 * D, 2 * D), lambda i: (0, 0))
    b_spec = pl.BlockSpec((1, 2 * D), lambda i: (0, 0))
    return pl.pallas_call(
        _main_body,
        grid=grid,
        in_specs=[row_spec, pk_spec, q_spec, q_spec,
                  w_spec, w_spec, w_spec, b_spec, b_spec],
        out_specs=q_spec,
        out_shape=jax.ShapeDtypeStruct((B, 2 * D), jnp.float32),
    )(ent_rows, trig_rows, anc, qtrig, w1at, w1bt, w2t, b1, b2)


@jax.jit
def kernel(entity_embedding, relation_embedding, W1, b1, W2, b2,
           anchors, rel_0, p1_target, p1_rel):
    rel_pad = jnp.zeros((TREL, D), jnp.float32).at[:N_REL + 1].set(
        relation_embedding)
    trig_table, trig_packed = _build_trig_tables(rel_pad)
    trig_packed = trig_packed.reshape(TREL * D)

    def pad_flat(idx):
        idx = jnp.pad(idx.astype(jnp.int32), ((0, 0), (0, K_PAD - K)))
        return idx.reshape(NW, NCHUNK, CHUNK)

    idx_ent = pad_flat(p1_target)
    idx_rel = pad_flat(p1_rel)
    idx_anc = anchors.astype(jnp.int32).reshape(NW, BPW)
    idx_rq = rel_0.astype(jnp.int32).reshape(NW, BPW)

    ent_rows, trig_flat, anc, qtrig = _sc_gather(
        entity_embedding, trig_table, trig_packed,
        idx_ent, idx_rel, idx_anc, idx_rq)
    trig_rows = trig_flat.reshape(ROWS, D)

    w1at = W1[:, :2 * D].T
    w1bt = W1[:, 2 * D:].T
    w2t = W2.T
    return _main_call(ent_rows, trig_rows, anc, qtrig,
                      w1at, w1bt, w2t,
                      b1.reshape(1, 2 * D), b2.reshape(1, 2 * D))


# speedup vs baseline: 1.2494x; 1.0021x over previous
import jax
import jax.numpy as jnp
from jax import lax
from jax.experimental import pallas as pl
from jax.experimental.pallas import tpu as pltpu
from jax.experimental.pallas import tpu_sc as plsc

PI = 3.141592653589793
N_ENT = 100000
N_REL = 1000
D = 64
B = 4096
K = 50
GAMMA = 24.0
EPS = 2.0
ER = (GAMMA + EPS) / D

K_PAD = 56
NC, NS = 2, 16
NW = NC * NS
ROWS = B * K_PAD
RPW = ROWS // NW
CHUNK = 112
NCHUNK = RPW // CHUNK
G = 4
SUP = NCHUNK // G
SROWS = G * CHUNK
BPW = B // NW
BB = 256
TREL = 1008
LANES = 16


def _trig_body(r_ref, out_ref, pk_ref):
    r = r_ref[...] * (PI / ER)
    c = jnp.cos(r)
    s = jnp.sin(r)
    out_ref[...] = jnp.concatenate([c, s], axis=-1)
    cb = lax.bitcast_convert_type(c, jnp.int32)
    sb = lax.bitcast_convert_type(s, jnp.int32)

    def rnd16(bits):
        return (bits + 0x7FFF + ((bits >> 16) & 1)) >> 16

    pk_ref[...] = (rnd16(sb) << 16) | (rnd16(cb) & 0xFFFF)


def _build_trig_tables(rel_pad):
    return pl.pallas_call(
        _trig_body,
        out_shape=(
            jax.ShapeDtypeStruct((TREL, 2 * D), jnp.float32),
            jax.ShapeDtypeStruct((TREL, D), jnp.int32),
        ),
    )(rel_pad)


def _sc_gather_body(ent_hbm, trig_hbm, trigpk_hbm,
                    idx_ent, idx_rel, idx_anc, idx_rq,
                    out_ent, out_trig, out_anc, out_rq,
                    idx_v, idx_s,
                    gsem_a, gsem_b, ssem_a, ssem_b, tsem_a, tsem_b, sem_s):
    wid = lax.axis_index("s") * NC + lax.axis_index("c")

    def smalls(buf):
        for idx_hbm, table, out in ((idx_anc, ent_hbm, out_anc),
                                    (idx_rq, trig_hbm, out_rq)):
            pltpu.sync_copy(idx_hbm.at[wid], idx_s)
            pltpu.async_copy(table.at[idx_s], buf, sem_s).wait()
            pltpu.sync_copy(buf, out.at[pl.ds(wid * BPW, BPW)])

    pl.run_scoped(smalls, pltpu.VMEM((BPW, 2 * D), jnp.float32))

    def big_phase(table_v, idx_vr, buf_a, buf_b, stg_a):
        pltpu.sync_copy(trigpk_hbm, table_v)
        pltpu.sync_copy(idx_ent.at[wid], idx_v)
        pltpu.sync_copy(idx_rel.at[wid], idx_vr)
        wbase = wid * RPW
        lanes = lax.iota(jnp.int32, LANES)

        def fire(j, buf, gsem):
            pltpu.async_copy(ent_hbm.at[idx_v.at[j]], buf, gsem)

        def drain_gather(buf, gsem):
            pltpu.make_async_copy(out_ent.at[pl.ds(0, CHUNK)], buf,
                                  gsem).wait()

        def scatter(j, buf, ssem):
            pltpu.async_copy(buf, out_ent.at[pl.ds(wbase + j * CHUNK, CHUNK)],
                             ssem)

        def wait_scatter(buf, ssem):
            pltpu.make_async_copy(buf, out_ent.at[pl.ds(wbase, CHUNK)],
                                  ssem).wait()

        def fill(j, stg):
            for g in range(CHUNK // LANES):
                rvec = idx_vr[j, pl.ds(g * LANES, LANES)] * D
                slot64 = (g * LANES + lanes) * D
                for w in range(D):
                    v = plsc.load_gather(table_v, [rvec + w])
                    plsc.store_scatter(stg, [slot64 + w], v)

        def wait_stg(stg, tsem):
            pltpu.make_async_copy(stg, out_trig.at[pl.ds(0, CHUNK * D)],
                                  tsem).wait()

        def scat_stg(j, stg, tsem):
            pltpu.async_copy(
                stg, out_trig.at[pl.ds((wbase + j * CHUNK) * D, CHUNK * D)],
                tsem)

        def step(j, buf, gsem, ssem, first, first_stg):
            if not first:
                wait_scatter(buf, ssem)
            fire(j, buf, gsem)
            if not first_stg:
                wait_stg(stg_a, tsem_a)
            fill(j, stg_a)
            scat_stg(j, stg_a, tsem_a)
            drain_gather(buf, gsem)
            scatter(j, buf, ssem)

        step(0, buf_a, gsem_a, ssem_a, True, True)
        step(1, buf_b, gsem_b, ssem_b, True, False)

        def body(j2, _):
            step(j2 * 2, buf_a, gsem_a, ssem_a, False, False)
            step(j2 * 2 + 1, buf_b, gsem_b, ssem_b, False, False)
            return 0

        lax.fori_loop(1, NCHUNK // 2, body, 0)
        wait_scatter(buf_a, ssem_a)
        wait_scatter(buf_b, ssem_b)
        wait_stg(stg_a, tsem_a)

    pl.run_scoped(big_phase,
                  pltpu.VMEM((TREL * D,), jnp.int32),
                  pltpu.VMEM((NCHUNK, CHUNK), jnp.int32),
                  pltpu.VMEM((CHUNK, 2 * D), jnp.float32),
                  pltpu.VMEM((CHUNK, 2 * D), jnp.float32),
                  pltpu.VMEM((CHUNK * D,), jnp.int32))


def _sc_gather(ent, trig, trigpk, idx_ent, idx_rel, idx_anc, idx_rq):
    mesh = plsc.VectorSubcoreMesh(
        core_axis_name="c", subcore_axis_name="s",
        num_cores=NC, num_subcores=NS)
    fn = pl.kernel(
        _sc_gather_body,
        compiler_params=pltpu.CompilerParams(needs_layout_passes=False),
        out_type=(
            jax.ShapeDtypeStruct((ROWS, 2 * D), jnp.float32),
            jax.ShapeDtypeStruct((ROWS * D,), jnp.int32),
            jax.ShapeDtypeStruct((B, 2 * D), jnp.float32),
            jax.ShapeDtypeStruct((B, 2 * D), jnp.float32),
        ),
        mesh=mesh,
        scratch_types=[
            pltpu.VMEM((NCHUNK, CHUNK), jnp.int32),
            pltpu.VMEM((BPW,), jnp.int32),
            pltpu.SemaphoreType.DMA,
            pltpu.SemaphoreType.DMA,
            pltpu.SemaphoreType.DMA,
            pltpu.SemaphoreType.DMA,
            pltpu.SemaphoreType.DMA,
            pltpu.SemaphoreType.DMA,
            pltpu.SemaphoreType.DMA,
        ],
    )
    return fn(ent, trig, trigpk, idx_ent, idx_rel, idx_anc, idx_rq)


def _main_body(ent_ref, trig_ref, anc_ref, qtrig_ref,
               w1at_ref, w1bt_ref, w2t_ref, b1_ref, b2_ref, out_ref):
    w = trig_ref[...]
    t_cos2 = lax.bitcast_convert_type(w << 16, jnp.float32)
    t_sin2 = lax.bitcast_convert_type((w >> 16) << 16, jnp.float32)
    trig2 = jnp.concatenate([t_cos2, t_sin2], axis=-1)
    tw = jnp.dot(trig2, w1bt_ref[...], preferred_element_type=jnp.float32)
    aq = jnp.dot(qtrig_ref[...], w1at_ref[...],
                 preferred_element_type=jnp.float32)
    act = jnp.maximum(
        tw.reshape(BB, K_PAD, 2 * D) + aq[:, None, :] + b1_ref[...][None],
        0.0)
    out2 = jnp.dot(act.reshape(BB * K_PAD, 2 * D), w2t_ref[...],
                   preferred_element_type=jnp.float32) + b2_ref[...]
    out2_3 = out2.reshape(BB, K_PAD, 2 * D)

    ccat = jnp.concatenate([t_cos2, t_cos2], axis=-1).reshape(BB, K_PAD, 2 * D)
    scat = jnp.concatenate([-t_sin2, t_sin2], axis=-1).reshape(BB, K_PAD, 2 * D)
    anc = anc_ref[...]
    swp = jnp.concatenate([anc[:, D:], anc[:, :D]], axis=-1)
    bias = (anc[:, None, :] * ccat + swp[:, None, :] * scat
            - ent_ref[...].reshape(BB, K_PAD, 2 * D))
    prod = out2_3 * bias
    kmask = lax.broadcasted_iota(jnp.int32, (1, K_PAD, 1), 1) < K
    fr = jnp.sum(jnp.where(kmask, prod, 0.0), axis=1) * (1.0 / K)

    q = qtrig_ref[...]
    qccat = jnp.concatenate([q[:, :D], q[:, :D]], axis=-1)
    qscat = jnp.concatenate([-q[:, D:], q[:, D:]], axis=-1)
    out_ref[...] = anc * qccat + swp * qscat + fr


def _main_call(ent_rows, trig_rows, anc, qtrig, w1at, w1bt, w2t, b1, b2):
    grid = (B // BB,)
    row_spec = pl.BlockSpec((BB * K_PAD, 2 * D), lambda i: (i, 0))
    pk_spec = pl.BlockSpec((BB * K_PAD, D), lambda i: (i, 0))
    q_spec = pl.BlockSpec((BB, 2 * D), lambda i: (i, 0))
    w_spec = pl.BlockSpec((2 * D, 2 * D), lambda i: (0, 0))
    b_spec = pl.BlockSpec((1, 2 * D), lambda i: (0, 0))
    return pl.pallas_call(
        _main_body,
        grid=grid,
        in_specs=[row_spec, pk_spec, q_spec, q_spec,
                  w_spec, w_spec, w_spec, b_spec, b_spec],
        out_specs=q_spec,
        out_shape=jax.ShapeDtypeStruct((B, 2 * D), jnp.float32),
    )(ent_rows, trig_rows, anc, qtrig, w1at, w1bt, w2t, b1, b2)


@jax.jit
def kernel(entity_embedding, relation_embedding, W1, b1, W2, b2,
           anchors, rel_0, p1_target, p1_rel):
    rel_pad = jnp.zeros((TREL, D), jnp.float32).at[:N_REL + 1].set(
        relation_embedding)
    trig_table, trig_packed = _build_trig_tables(rel_pad)
    trig_packed = trig_packed.reshape(TREL * D)

    def pad_flat(idx):
        idx = jnp.pad(idx.astype(jnp.int32), ((0, 0), (0, K_PAD - K)))
        return idx.reshape(NW, NCHUNK, CHUNK)

    idx_ent = pad_flat(p1_target)
    idx_rel = pad_flat(p1_rel)
    idx_anc = anchors.astype(jnp.int32).reshape(NW, BPW)
    idx_rq = rel_0.astype(jnp.int32).reshape(NW, BPW)

    ent_rows, trig_flat, anc, qtrig = _sc_gather(
        entity_embedding, trig_table, trig_packed,
        idx_ent, idx_rel, idx_anc, idx_rq)
    trig_rows = trig_flat.reshape(ROWS, D)

    w1at = W1[:, :2 * D].T
    w1bt = W1[:, 2 * D:].T
    w2t = W2.T
    return _main_call(ent_rows, trig_rows, anc, qtrig,
                      w1at, w1bt, w2t,
                      b1.reshape(1, 2 * D), b2.reshape(1, 2 * D))
